# Initial kernel scaffold; baseline (speedup 1.0000x reference)
import jax
import jax.numpy as jnp
from jax.experimental import pallas as pl


def kernel(value, key, query, Wv, Wk, Wq, Wo, bo):
    raise NotImplementedError("write your pallas kernel here")


# speedup vs baseline: 12441.6551x; 12441.6551x over previous
import functools

import jax
import jax.numpy as jnp
from jax.experimental import pallas as pl

_H = 16
_M = 256


def _proj_kernel(x_ref, w_ref, o_ref):
    o_ref[0] = jax.lax.dot_general(
        x_ref[...], w_ref[...], (((1,), (1,)), ((), ())),
        preferred_element_type=jnp.float32)


def _project(x, W, N, S, H, D, Br):
    NS, E = N * S, H * D
    sb = S // Br
    return pl.pallas_call(
        _proj_kernel,
        grid=(NS // Br, H),
        in_specs=[
            pl.BlockSpec((Br, E), lambda i, h: (i, 0)),
            pl.BlockSpec((D, E), lambda i, h: (h, 0)),
        ],
        out_specs=pl.BlockSpec(
            (1, Br, D), lambda i, h: ((i // sb) * H + h, i % sb, 0)),
        out_shape=jax.ShapeDtypeStruct((N * H, S, D), jnp.float32),
    )(x.reshape(NS, E), W)


def _attn_kernel(q_ref, k_ref, v_ref, o_ref, *, M):
    q = q_ref[0]
    k = k_ref[0]
    v = v_ref[0]
    e = jax.lax.dot_general(
        q, k, (((1,), (1,)), ((), ())), preferred_element_type=jnp.float32)
    S = e.shape[1]
    bits = jax.lax.bitcast_convert_type(e, jnp.int32)
    flip = (bits >> 31) | jnp.int32(-2147483648)
    u15 = jax.lax.shift_right_logical(bits ^ flip, 17)
    packed = (u15[:, : S // 2] | (u15[:, S // 2 :] << 16)) | jnp.int32(
        -2147450880)
    def body(_, c):
        lo, hi = c
        mid = (lo + hi) >> 1
        midp = mid | (mid << 16)
        sub = packed - midp
        inc = jax.lax.shift_right_logical(sub, 15) & jnp.int32(0x00010001)
        acc = jnp.sum(inc, axis=1, keepdims=True)
        cnt = (acc & jnp.int32(0xFFFF)) + jax.lax.shift_right_logical(acc, 16)
        ge = cnt >= M
        return jnp.where(ge, mid, lo), jnp.where(ge, hi, mid)
    lo, _ = jax.lax.fori_loop(0, 15, body,
                              (jnp.zeros((e.shape[0], 1), jnp.int32),
                               jnp.full((e.shape[0], 1), 32768, jnp.int32)))
    sel = u15 >= lo
    emax = jnp.max(e, axis=1, keepdims=True)
    p = jnp.where(sel, jnp.exp(e - emax), 0.0)
    inv = 1.0 / jnp.sum(p, axis=1, keepdims=True)
    o_ref[0] = jax.lax.dot_general(
        p, v, (((1,), (0,)), ((), ())), preferred_element_type=jnp.float32) * inv


def _attention(q3, k3, v3, NH, S, D, Bq, M):
    return pl.pallas_call(
        functools.partial(_attn_kernel, M=M),
        grid=(NH, S // Bq),
        in_specs=[
            pl.BlockSpec((1, Bq, D), lambda h, i: (h, i, 0)),
            pl.BlockSpec((1, S, D), lambda h, i: (h, 0, 0)),
            pl.BlockSpec((1, S, D), lambda h, i: (h, 0, 0)),
        ],
        out_specs=pl.BlockSpec((1, Bq, D), lambda h, i: (h, i, 0)),
        out_shape=jax.ShapeDtypeStruct((NH, S, D), jnp.float32),
    )(q3, k3, v3)


def _out_kernel(x_ref, w_ref, b_ref, o_ref):
    o_ref[...] = jax.lax.dot_general(
        x_ref[...], w_ref[...], (((1,), (1,)), ((), ())),
        preferred_element_type=jnp.float32) + b_ref[...]


def _out_project(y, Wo, bo, NS, E, Br):
    return pl.pallas_call(
        _out_kernel,
        grid=(NS // Br,),
        in_specs=[
            pl.BlockSpec((Br, E), lambda i: (i, 0)),
            pl.BlockSpec((E, E), lambda i: (0, 0)),
            pl.BlockSpec((1, E), lambda i: (0, 0)),
        ],
        out_specs=pl.BlockSpec((Br, E), lambda i: (i, 0)),
        out_shape=jax.ShapeDtypeStruct((NS, E), jnp.float32),
    )(y, Wo, bo.reshape(1, E))


def kernel(value, key, query, Wv, Wk, Wq, Wo, bo):
    N, S, E = query.shape
    H = _H
    D = E // H
    M = _M
    Br = min(512, S)
    Bq = min(1024, S)
    q3 = _project(query, Wq, N, S, H, D, Br)
    k3 = _project(key, Wk, N, S, H, D, Br)
    v3 = _project(value, Wv, N, S, H, D, Br)
    out4 = _attention(q3, k3, v3, N * H, S, D, Bq, M)
    y = out4.reshape(N * S, E)
    return _out_project(y, Wo, bo, N * S, E, Br).reshape(N, S, E)


# speedup vs baseline: 13487.0388x; 1.0840x over previous
import functools

import jax
import jax.numpy as jnp
from jax.experimental import pallas as pl

_H = 16
_M = 256


def _proj_kernel(xq_ref, xk_ref, xv_ref, wq_ref, wk_ref, wv_ref,
                 oq_ref, ok_ref, ov_ref):
    for x_ref, w_ref, o_ref in ((xq_ref, wq_ref, oq_ref),
                                (xk_ref, wk_ref, ok_ref),
                                (xv_ref, wv_ref, ov_ref)):
        o_ref[0] = jax.lax.dot_general(
            x_ref[...], w_ref[...], (((1,), (1,)), ((), ())),
            preferred_element_type=jnp.float32)


def _project_qkv(xq, xk, xv, Wq, Wk, Wv, N, S, H, D, Br):
    NS, E = N * S, H * D
    sb = S // Br
    xspec = pl.BlockSpec((Br, E), lambda i, h: (i, 0))
    wspec = pl.BlockSpec((D, E), lambda i, h: (h, 0))
    ospec = pl.BlockSpec(
        (1, Br, D), lambda i, h: ((i // sb) * H + h, i % sb, 0))
    oshape = jax.ShapeDtypeStruct((N * H, S, D), jnp.float32)
    return pl.pallas_call(
        _proj_kernel,
        grid=(NS // Br, H),
        in_specs=[xspec, xspec, xspec, wspec, wspec, wspec],
        out_specs=[ospec, ospec, ospec],
        out_shape=[oshape, oshape, oshape],
    )(xq.reshape(NS, E), xk.reshape(NS, E), xv.reshape(NS, E), Wq, Wk, Wv)


def _attn_kernel(q_ref, k_ref, v_ref, o_ref, *, M):
    q = q_ref[0]
    k = k_ref[0]
    v = v_ref[0]
    e = jax.lax.dot_general(
        q, k, (((1,), (1,)), ((), ())), preferred_element_type=jnp.float32)
    S = e.shape[1]
    bits = jax.lax.bitcast_convert_type(e, jnp.int32)
    flip = (bits >> 31) | jnp.int32(-2147483648)
    u15 = jax.lax.shift_right_logical(bits ^ flip, 17)
    packed = (u15[:, : S // 2] | (u15[:, S // 2 :] << 16)) | jnp.int32(
        -2147450880)
    def body(_, c):
        lo, hi = c
        mid = (lo + hi) >> 1
        midp = mid | (mid << 16)
        sub = packed - midp
        inc = jax.lax.shift_right_logical(sub, 15) & jnp.int32(0x00010001)
        acc = jnp.sum(inc, axis=1, keepdims=True)
        cnt = (acc & jnp.int32(0xFFFF)) + jax.lax.shift_right_logical(acc, 16)
        ge = cnt >= M
        return jnp.where(ge, mid, lo), jnp.where(ge, hi, mid)
    lo, _ = jax.lax.fori_loop(0, 15, body,
                              (jnp.zeros((e.shape[0], 1), jnp.int32),
                               jnp.full((e.shape[0], 1), 32768, jnp.int32)))
    sel = u15 >= lo
    emax = jnp.max(e, axis=1, keepdims=True)
    p = jnp.where(sel, jnp.exp(e - emax), 0.0)
    inv = 1.0 / jnp.sum(p, axis=1, keepdims=True)
    o_ref[0] = jax.lax.dot_general(
        p, v, (((1,), (0,)), ((), ())), preferred_element_type=jnp.float32) * inv


def _attention(q3, k3, v3, NH, S, D, Bq, M):
    return pl.pallas_call(
        functools.partial(_attn_kernel, M=M),
        grid=(NH, S // Bq),
        in_specs=[
            pl.BlockSpec((1, Bq, D), lambda h, i: (h, i, 0)),
            pl.BlockSpec((1, S, D), lambda h, i: (h, 0, 0)),
            pl.BlockSpec((1, S, D), lambda h, i: (h, 0, 0)),
        ],
        out_specs=pl.BlockSpec((1, Bq, D), lambda h, i: (h, i, 0)),
        out_shape=jax.ShapeDtypeStruct((NH, S, D), jnp.float32),
    )(q3, k3, v3)


def _out_kernel(x_ref, w_ref, b_ref, o_ref):
    o_ref[...] = jax.lax.dot_general(
        x_ref[...], w_ref[...], (((1,), (1,)), ((), ())),
        preferred_element_type=jnp.float32) + b_ref[...]


def _out_project(y, Wo, bo, NS, E, Br):
    return pl.pallas_call(
        _out_kernel,
        grid=(NS // Br,),
        in_specs=[
            pl.BlockSpec((Br, E), lambda i: (i, 0)),
            pl.BlockSpec((E, E), lambda i: (0, 0)),
            pl.BlockSpec((1, E), lambda i: (0, 0)),
        ],
        out_specs=pl.BlockSpec((Br, E), lambda i: (i, 0)),
        out_shape=jax.ShapeDtypeStruct((NS, E), jnp.float32),
    )(y, Wo, bo.reshape(1, E))


def kernel(value, key, query, Wv, Wk, Wq, Wo, bo):
    N, S, E = query.shape
    H = _H
    D = E // H
    M = _M
    Br = min(512, S)
    Bq = min(1024, S)
    q3, k3, v3 = _project_qkv(query, key, value, Wq, Wk, Wv,
                              N, S, H, D, Br)
    out4 = _attention(q3, k3, v3, N * H, S, D, Bq, M)
    y = out4.reshape(N * S, E)
    return _out_project(y, Wo, bo, N * S, E, Br).reshape(N, S, E)
